# baseline (speedup 1.0000x reference)
import functools

import numpy as np
import jax
import jax.numpy as jnp
from jax.experimental import pallas as pl
from jax.experimental.pallas import tpu as pltpu

_VMEM_LIMIT_BYTES = 100 * 1024 * 1024


def _mean_nh_kernel(x_ref, m_ref, o_ref, acc_ref, *, inv_count):
    ni = pl.program_id(1)
    nn = pl.num_programs(1)

    @pl.when(ni == 0)
    def _init():
        acc_ref[...] = jnp.zeros_like(acc_ref)

    acc_ref[...] += jnp.sum(x_ref[...], axis=0, dtype=jnp.float32)

    @pl.when(ni == nn - 1)
    def _finish():
        folded = jax.lax.dot(
            acc_ref[...],
            m_ref[...],
            precision=jax.lax.Precision.HIGHEST,
            preferred_element_type=jnp.float32,
        )
        o_ref[...] = (folded * inv_count).astype(o_ref.dtype)


def _largest_divisor_at_most(n: int, cap: int) -> int:
    for cand in range(min(cap, n), 0, -1):
        if n % cand == 0:
            return cand
    return 1


def kernel(x):
    n, c, h, w = x.shape
    hw = h * w
    inv_count = 1.0 / float(n * h)

    x3 = x.reshape(n, c, hw)

    m = (jnp.arange(hw, dtype=jnp.int32)[:, None] % w
         == jnp.arange(w, dtype=jnp.int32)[None, :]).astype(jnp.float32)

    c_blk = c // 2 if (c % 2 == 0 and (c // 2) % 8 == 0) else c
    itemsize = np.dtype(x.dtype).itemsize
    row_bytes = c_blk * ((hw + 127) // 128 * 128) * itemsize
    n_blk = _largest_divisor_at_most(n, max(1, (14 * 1024 * 1024) // row_bytes))

    grid = (c // c_blk, n // n_blk)

    out = pl.pallas_call(
        functools.partial(_mean_nh_kernel, inv_count=inv_count),
        out_shape=jax.ShapeDtypeStruct((c, w), x.dtype),
        grid=grid,
        in_specs=[
            pl.BlockSpec((n_blk, c_blk, hw), lambda ci, ni: (ni, ci, 0)),
            pl.BlockSpec((hw, w), lambda ci, ni: (0, 0)),
        ],
        out_specs=pl.BlockSpec((c_blk, w), lambda ci, ni: (ci, 0)),
        scratch_shapes=[pltpu.VMEM((c_blk, hw), jnp.float32)],
        compiler_params=pltpu.CompilerParams(
            dimension_semantics=("parallel", "arbitrary"),
            vmem_limit_bytes=_VMEM_LIMIT_BYTES,
        ),
        cost_estimate=pl.CostEstimate(
            flops=n * c * hw + c * hw * w * 2,
            transcendentals=0,
            bytes_accessed=n * c * hw * itemsize + c * w * itemsize,
        ),
    )(x3, m)
    return out
